# baseline (speedup 1.0000x reference)
import functools

import jax
import jax.numpy as jnp
from jax import lax
from jax.experimental import pallas as pl
from jax.experimental.pallas import tpu as pltpu
from jax.experimental.pallas import tpu_sc as plsc

F32 = jnp.float32


def _node_mm_body(x_ref, w_ref, b_ref, o_ref):
    o_ref[...] = (jnp.dot(x_ref[...], w_ref[...], preferred_element_type=F32)
                  + b_ref[0:1, :])


def _node_mm(x, w, b8):
    N, _ = x.shape
    K = w.shape[1]
    return pl.pallas_call(
        _node_mm_body,
        out_shape=jax.ShapeDtypeStruct((N, K), F32),
    )(x, w, b8)


def _unpack_i32(x):
    lo = jax.lax.bitcast_convert_type(x << 16, F32)
    hi = jax.lax.bitcast_convert_type(x & jnp.int32(-65536), F32)
    return jnp.concatenate([lo, hi], axis=1)


def _edge1_body(e_ref, ga_ref, gb_ref, bs_ref, w_ref, p_ref,
                t_ref, sig_ref, sb_ref, st_ref):
    t = (jnp.dot(e_ref[...], w_ref[...], preferred_element_type=F32)
         + p_ref[0:1, :] + _unpack_i32(ga_ref[...]) + _unpack_i32(gb_ref[...]))
    t_ref[...] = t.astype(t_ref.dtype)
    sig = 1.0 / (1.0 + jnp.exp(-t))
    sig_ref[...] = sig
    sb_ref[...] = sig * _unpack_i32(bs_ref[...])

    @pl.when(pl.program_id(0) == 0)
    def _():
        st_ref[...] = jnp.zeros_like(st_ref)

    st_ref[0:1, :] += jnp.sum(t, axis=0, keepdims=True)
    st_ref[1:2, :] += jnp.sum(t * t, axis=0, keepdims=True)


def _edge1(e0, ga, gb, bsrc, w, p8, eb):
    E, D = e0.shape
    blk = pl.BlockSpec((eb, D), lambda i: (i, 0))
    blkh = pl.BlockSpec((eb, D // 2), lambda i: (i, 0))
    full = pl.BlockSpec((8, D), lambda i: (0, 0))
    wspec = pl.BlockSpec((D, D), lambda i: (0, 0))
    return pl.pallas_call(
        _edge1_body,
        grid=(E // eb,),
        in_specs=[blk, blkh, blkh, blkh, wspec, full],
        out_specs=[blk, blk, blk, full],
        out_shape=[jax.ShapeDtypeStruct((E, D), jnp.bfloat16),
                   jax.ShapeDtypeStruct((E, D), F32),
                   jax.ShapeDtypeStruct((E, D), F32),
                   jax.ShapeDtypeStruct((8, D), F32)],
    )(e0, ga, gb, bsrc, w, p8)


def _node_fin_body(h0_ref, ah_ref, num_ref, den_ref, pg_ref, w_ref, b_ref, o_ref):
    hpre = ah_ref[...] + num_ref[...] / (den_ref[...] + 1e-6)
    mu = jnp.mean(hpre, axis=0, keepdims=True)
    d = hpre - mu
    var = jnp.mean(d * d, axis=0, keepdims=True)
    hn = pg_ref[0:1, :] * d / jnp.sqrt(var + 1e-5) + pg_ref[1:2, :]
    h1 = h0_ref[...] + jnp.maximum(hn, 0.0)
    o_ref[...] = (jnp.dot(h1, w_ref[...], preferred_element_type=F32)
                  + b_ref[0:1, :])


def _node_fin(h0, ah, num, den, pg8, w2, b2):
    N, _ = h0.shape
    K = w2.shape[1]
    return pl.pallas_call(
        _node_fin_body,
        out_shape=jax.ShapeDtypeStruct((N, K), F32),
    )(h0, ah, num, den, pg8, w2, b2)


def _edge2_body(inv_e, e0_ref, t1_ref, ga_ref, gb_ref, st1_ref, pp_ref, w_ref,
                e1_ref, t2_ref, st2_ref):
    mu = st1_ref[0:1, :] * inv_e
    var = st1_ref[1:2, :] * inv_e - mu * mu
    en = (pp_ref[0:1, :] * (t1_ref[...].astype(F32) - mu) / jnp.sqrt(var + 1e-5)
          + pp_ref[1:2, :])
    e1 = e0_ref[...] + jnp.maximum(en, 0.0)
    e1_ref[...] = e1
    t2 = (jnp.dot(e1, w_ref[...], preferred_element_type=F32)
          + pp_ref[2:3, :] + _unpack_i32(ga_ref[...]) + _unpack_i32(gb_ref[...]))
    t2_ref[...] = t2.astype(t2_ref.dtype)

    @pl.when(pl.program_id(0) == 0)
    def _():
        st2_ref[...] = jnp.zeros_like(st2_ref)

    st2_ref[0:1, :] += jnp.sum(t2, axis=0, keepdims=True)
    st2_ref[1:2, :] += jnp.sum(t2 * t2, axis=0, keepdims=True)


def _edge2(e0, t1, ga, gb, st1, pp8, w, eb):
    E, D = e0.shape
    blk = pl.BlockSpec((eb, D), lambda i: (i, 0))
    blkh = pl.BlockSpec((eb, D // 2), lambda i: (i, 0))
    full = pl.BlockSpec((8, D), lambda i: (0, 0))
    wspec = pl.BlockSpec((D, D), lambda i: (0, 0))
    return pl.pallas_call(
        functools.partial(_edge2_body, 1.0 / E),
        grid=(E // eb,),
        in_specs=[blk, blk, blkh, blkh, full, full, wspec],
        out_specs=[blk, blk, full],
        out_shape=[jax.ShapeDtypeStruct((E, D), F32),
                   jax.ShapeDtypeStruct((E, D), jnp.bfloat16),
                   jax.ShapeDtypeStruct((8, D), F32)],
    )(e0, t1, ga, gb, st1, pp8, w)


def _edge3_body(inv_e, e1_ref, t2_ref, st2_ref, pp_ref, o_ref):
    mu = st2_ref[0:1, :] * inv_e
    var = st2_ref[1:2, :] * inv_e - mu * mu
    en = (pp_ref[0:1, :] * (t2_ref[...].astype(F32) - mu) / jnp.sqrt(var + 1e-5)
          + pp_ref[1:2, :])
    o_ref[...] = e1_ref[...] + jnp.maximum(en, 0.0)


def _edge3(e1, t2, st2, pp8, eb):
    E, D = e1.shape
    blk = pl.BlockSpec((eb, D), lambda i: (i, 0))
    full = pl.BlockSpec((8, D), lambda i: (0, 0))
    return pl.pallas_call(
        functools.partial(_edge3_body, 1.0 / E),
        grid=(E // eb,),
        in_specs=[blk, blk, full, full],
        out_specs=blk,
        out_shape=jax.ShapeDtypeStruct((E, D), F32),
    )(e1, t2, st2, pp8)


def _sc_gather(tables, idx_sel, eidx, W=400):
    E = eidx[0].shape[0]
    N, D = tables[0].shape
    dt = tables[0].dtype
    info = plsc.get_sparse_core_info()
    NC, NS = info.num_cores, info.num_subcores
    NW = NC * NS
    PW = E // NW
    nwin = PW // W
    assert PW % W == 0 and W % 8 == 0
    nt = len(tables)
    mesh = plsc.VectorSubcoreMesh(core_axis_name="c", subcore_axis_name="s")

    @functools.partial(
        pl.kernel,
        out_type=[jax.ShapeDtypeStruct((E, D), dt) for _ in range(nt)],
        mesh=mesh,
        compiler_params=pltpu.CompilerParams(use_tc_tiling_on_sc=False),
        scratch_types=(
            [pltpu.VMEM((W,), jnp.int32), pltpu.VMEM((W,), jnp.int32)]
            + [pltpu.VMEM((W, D), dt) for _ in range(nt)]
            + [pltpu.SemaphoreType.DMA, pltpu.SemaphoreType.DMA,
               pltpu.SemaphoreType.DMA]
        ),
    )
    def k(*refs):
        tabs = refs[:nt]
        src_hbm, dst_hbm = refs[nt], refs[nt + 1]
        outs = refs[nt + 2:2 * nt + 2]
        sidx, didx = refs[2 * nt + 2], refs[2 * nt + 3]
        rows = refs[2 * nt + 4:3 * nt + 4]
        sem_i, sem_g, sem_o = refs[3 * nt + 4:3 * nt + 7]
        wid = lax.axis_index("s") * NC + lax.axis_index("c")

        @pl.loop(0, nwin)
        def _(w):
            base = wid * PW + w * W
            di = [pltpu.async_copy(src_hbm.at[pl.ds(base, W)], sidx, sem_i),
                  pltpu.async_copy(dst_hbm.at[pl.ds(base, W)], didx, sem_i)]
            for d in di:
                d.wait()
            ds = [pltpu.async_copy(
                tabs[t].at[sidx if idx_sel[t] == 0 else didx], rows[t],
                sem_g) for t in range(nt)]
            for d in ds:
                d.wait()
            os_ = [pltpu.async_copy(rows[t], outs[t].at[pl.ds(base, W)], sem_o)
                   for t in range(nt)]
            for d in os_:
                d.wait()

    return k(*tables, eidx[0], eidx[1])


def _sc_segsum2(sb, sig, dst, N, W=200):
    E, D = sb.shape
    info = plsc.get_sparse_core_info()
    NS = info.num_subcores
    PW = E // NS
    nwin = PW // W
    assert PW % W == 0 and W % 8 == 0
    zeros = jnp.zeros((N, D), F32)
    mesh = plsc.VectorSubcoreMesh(core_axis_name="c", subcore_axis_name="s")

    @functools.partial(
        pl.kernel,
        out_type=jax.ShapeDtypeStruct((2, N, D), F32),
        mesh=mesh,
        scratch_types=[
            pltpu.VMEM((W,), jnp.int32),
            pltpu.VMEM((W, D), F32),
            pltpu.VMEM_SHARED((N, D), F32),
        ],
    )
    def k(sb_hbm, sig_hbm, dst_hbm, z_hbm, out_hbm, idxb, upd, acc):
        c = lax.axis_index("c")
        s = lax.axis_index("s")

        @pl.when(s == 0)
        def _():
            pltpu.sync_copy(z_hbm, acc)

        plsc.subcore_barrier()

        @pl.loop(0, nwin)
        def _(w):
            base = s * PW + w * W
            pltpu.sync_copy(dst_hbm.at[pl.ds(base, W)], idxb)

            @pl.when(c == 0)
            def _():
                pltpu.sync_copy(sb_hbm.at[pl.ds(base, W)], upd)

            @pl.when(c == 1)
            def _():
                pltpu.sync_copy(sig_hbm.at[pl.ds(base, W)], upd)

            pltpu.sync_copy(upd, acc.at[idxb], add=True)

        plsc.subcore_barrier()

        @pl.when(s == 0)
        def _():
            pltpu.sync_copy(acc, out_hbm.at[c])

    return k(sb, sig, dst, zeros)


def _pack_i32(x):
    d = x.shape[1]
    xb = x.astype(jnp.bfloat16)
    u = jax.lax.bitcast_convert_type(xb, jnp.uint16).astype(jnp.uint32)
    w = (u[:, d // 2:] << 16) | u[:, :d // 2]
    return jax.lax.bitcast_convert_type(w, jnp.int32)


def _pack8(D, *rows):
    a = jnp.stack(rows)
    return jnp.concatenate([a, jnp.zeros((8 - a.shape[0], D), F32)], axis=0)


def kernel(h, e, params, edge_index):
    N, D = h.shape
    src = edge_index[0]
    dst = edge_index[1]
    p1, p2 = params
    eb = 4000

    w1 = jnp.concatenate([p1['A_w'], p1['B_w'], p1['D_w'], p1['E_w']], axis=1)
    b1 = jnp.concatenate([p1['A_b'], p1['B_b'], p1['D_b'], p1['E_b']])
    node1 = _node_mm(h, w1, _pack8(4 * D, b1))
    ah = node1[:, :D]
    bh = _pack_i32(node1[:, D:2 * D])
    dh = _pack_i32(node1[:, 2 * D:3 * D])
    eh = _pack_i32(node1[:, 3 * D:])

    ga1, gb1, bsrc = _sc_gather((dh, eh, bh), (0, 1, 0), edge_index)

    t1, sig, sb, st1 = _edge1(e, ga1, gb1, bsrc, p1['C_w'],
                              _pack8(D, p1['C_b']), eb)

    accs = _sc_segsum2(sb, sig, dst, N)
    num, den = accs[0], accs[1]

    w2 = jnp.concatenate([p2['D_w'], p2['E_w']], axis=1)
    b2 = jnp.concatenate([p2['D_b'], p2['E_b']])
    node2 = _node_fin(h, ah, num, den, _pack8(D, p1['bn_h_g'], p1['bn_h_b']),
                      w2, _pack8(2 * D, b2))
    dh2 = _pack_i32(node2[:, :D])
    eh2 = _pack_i32(node2[:, D:])

    ga2, gb2 = _sc_gather((dh2, eh2), (0, 1), edge_index)

    e1, t2, st2 = _edge2(e, t1, ga2, gb2, st1,
                         _pack8(D, p1['bn_e_g'], p1['bn_e_b'], p2['C_b']),
                         p2['C_w'], eb)

    e2 = _edge3(e1, t2, st2, _pack8(D, p2['bn_e_g'], p2['bn_e_b']), eb)

    return (h, e2)


# speedup vs baseline: 1.2065x; 1.2065x over previous
import functools

import jax
import jax.numpy as jnp
from jax import lax
from jax.experimental import pallas as pl
from jax.experimental.pallas import tpu as pltpu
from jax.experimental.pallas import tpu_sc as plsc

F32 = jnp.float32


def _node_mm_body(x_ref, w_ref, b_ref, o_ref):
    o_ref[...] = (jnp.dot(x_ref[...], w_ref[...], preferred_element_type=F32)
                  + b_ref[0:1, :])


def _node_mm(x, w, b8):
    N, _ = x.shape
    K = w.shape[1]
    return pl.pallas_call(
        _node_mm_body,
        out_shape=jax.ShapeDtypeStruct((N, K), F32),
    )(x, w, b8)


def _edge1_body(e_ref, ga_ref, gb_ref, bs_ref, w_ref, p_ref,
                t_ref, sig_ref, sb_ref, st_ref):
    t = (jnp.dot(e_ref[...], w_ref[...], preferred_element_type=F32)
         + p_ref[0:1, :] + ga_ref[...] + gb_ref[...])
    t_ref[...] = t.astype(t_ref.dtype)
    sig = 1.0 / (1.0 + jnp.exp(-t))
    sig_ref[...] = sig
    sb_ref[...] = sig * bs_ref[...]

    @pl.when(pl.program_id(0) == 0)
    def _():
        st_ref[...] = jnp.zeros_like(st_ref)

    st_ref[0:1, :] += jnp.sum(t, axis=0, keepdims=True)
    st_ref[1:2, :] += jnp.sum(t * t, axis=0, keepdims=True)


def _edge1(e0, ga, gb, bsrc, w, p8, eb):
    E, D = e0.shape
    blk = pl.BlockSpec((eb, D), lambda i: (i, 0))
    full = pl.BlockSpec((8, D), lambda i: (0, 0))
    wspec = pl.BlockSpec((D, D), lambda i: (0, 0))
    return pl.pallas_call(
        _edge1_body,
        grid=(E // eb,),
        in_specs=[blk, blk, blk, blk, wspec, full],
        out_specs=[blk, blk, blk, full],
        out_shape=[jax.ShapeDtypeStruct((E, D), jnp.bfloat16),
                   jax.ShapeDtypeStruct((E, D), F32),
                   jax.ShapeDtypeStruct((E, D), F32),
                   jax.ShapeDtypeStruct((8, D), F32)],
    )(e0, ga, gb, bsrc, w, p8)


def _node_fin_body(h0_ref, ah_ref, num_ref, den_ref, pg_ref, w_ref, b_ref, o_ref):
    hpre = ah_ref[...] + num_ref[...] / (den_ref[...] + 1e-6)
    mu = jnp.mean(hpre, axis=0, keepdims=True)
    d = hpre - mu
    var = jnp.mean(d * d, axis=0, keepdims=True)
    hn = pg_ref[0:1, :] * d / jnp.sqrt(var + 1e-5) + pg_ref[1:2, :]
    h1 = h0_ref[...] + jnp.maximum(hn, 0.0)
    o_ref[...] = (jnp.dot(h1, w_ref[...], preferred_element_type=F32)
                  + b_ref[0:1, :])


def _node_fin(h0, ah, num, den, pg8, w2, b2):
    N, _ = h0.shape
    K = w2.shape[1]
    return pl.pallas_call(
        _node_fin_body,
        out_shape=jax.ShapeDtypeStruct((N, K), F32),
    )(h0, ah, num, den, pg8, w2, b2)


def _e1pass_body(inv_e, e0_ref, t1_ref, st1_ref, pp_ref, e1_ref):
    mu = st1_ref[0:1, :] * inv_e
    var = st1_ref[1:2, :] * inv_e - mu * mu
    en = (pp_ref[0:1, :] * (t1_ref[...].astype(F32) - mu) / jnp.sqrt(var + 1e-5)
          + pp_ref[1:2, :])
    e1_ref[...] = e0_ref[...] + jnp.maximum(en, 0.0)


def _e1pass(e0, t1, st1, pp8, eb):
    E, D = e0.shape
    blk = pl.BlockSpec((eb, D), lambda i: (i, 0))
    full = pl.BlockSpec((8, D), lambda i: (0, 0))
    return pl.pallas_call(
        functools.partial(_e1pass_body, 1.0 / E),
        grid=(E // eb,),
        in_specs=[blk, blk, full, full],
        out_specs=blk,
        out_shape=jax.ShapeDtypeStruct((E, D), F32),
    )(e0, t1, st1, pp8)


def _t2pass_body(e1_ref, ga_ref, gb_ref, pp_ref, w_ref, t2_ref, st2_ref):
    t2 = (jnp.dot(e1_ref[...], w_ref[...], preferred_element_type=F32)
          + pp_ref[2:3, :] + ga_ref[...] + gb_ref[...])
    t2_ref[...] = t2.astype(t2_ref.dtype)

    @pl.when(pl.program_id(0) == 0)
    def _():
        st2_ref[...] = jnp.zeros_like(st2_ref)

    st2_ref[0:1, :] += jnp.sum(t2, axis=0, keepdims=True)
    st2_ref[1:2, :] += jnp.sum(t2 * t2, axis=0, keepdims=True)


def _t2pass(e1, ga, gb, pp8, w, eb):
    E, D = e1.shape
    blk = pl.BlockSpec((eb, D), lambda i: (i, 0))
    full = pl.BlockSpec((8, D), lambda i: (0, 0))
    wspec = pl.BlockSpec((D, D), lambda i: (0, 0))
    return pl.pallas_call(
        _t2pass_body,
        grid=(E // eb,),
        in_specs=[blk, blk, blk, full, wspec],
        out_specs=[blk, full],
        out_shape=[jax.ShapeDtypeStruct((E, D), jnp.bfloat16),
                   jax.ShapeDtypeStruct((8, D), F32)],
    )(e1, ga, gb, pp8, w)


def _edge3_body(inv_e, e1_ref, t2_ref, st2_ref, pp_ref, o_ref):
    mu = st2_ref[0:1, :] * inv_e
    var = st2_ref[1:2, :] * inv_e - mu * mu
    en = (pp_ref[0:1, :] * (t2_ref[...].astype(F32) - mu) / jnp.sqrt(var + 1e-5)
          + pp_ref[1:2, :])
    o_ref[...] = e1_ref[...] + jnp.maximum(en, 0.0)


def _edge3(e1, t2, st2, pp8, eb):
    E, D = e1.shape
    blk = pl.BlockSpec((eb, D), lambda i: (i, 0))
    full = pl.BlockSpec((8, D), lambda i: (0, 0))
    return pl.pallas_call(
        functools.partial(_edge3_body, 1.0 / E),
        grid=(E // eb,),
        in_specs=[blk, blk, full, full],
        out_specs=blk,
        out_shape=jax.ShapeDtypeStruct((E, D), F32),
    )(e1, t2, st2, pp8)


def _sc_gather(tables, idx_sel, eidx, W=None):
    E = eidx[0].shape[0]
    N, D = tables[0].shape
    dt = tables[0].dtype
    nt = len(tables)
    if W is None:
        W = 400 if nt <= 2 else 200
    info = plsc.get_sparse_core_info()
    NC, NS = info.num_cores, info.num_subcores
    NW = NC * NS
    PW = E // NW
    nwin = PW // W
    assert PW % W == 0 and W % 8 == 0
    mesh = plsc.VectorSubcoreMesh(core_axis_name="c", subcore_axis_name="s")

    @functools.partial(
        pl.kernel,
        out_type=[jax.ShapeDtypeStruct((E, D), dt) for _ in range(nt)],
        mesh=mesh,
        scratch_types=(
            [pltpu.VMEM((W,), jnp.int32), pltpu.VMEM((W,), jnp.int32)]
            + [pltpu.VMEM((W, D), dt) for _ in range(nt)]
            + [pltpu.SemaphoreType.DMA, pltpu.SemaphoreType.DMA,
               pltpu.SemaphoreType.DMA]
        ),
    )
    def k(*refs):
        tabs = refs[:nt]
        src_hbm, dst_hbm = refs[nt], refs[nt + 1]
        outs = refs[nt + 2:2 * nt + 2]
        sidx, didx = refs[2 * nt + 2], refs[2 * nt + 3]
        rows = refs[2 * nt + 4:3 * nt + 4]
        sem_i, sem_g, sem_o = refs[3 * nt + 4:3 * nt + 7]
        wid = lax.axis_index("s") * NC + lax.axis_index("c")

        @pl.loop(0, nwin)
        def _(w):
            base = wid * PW + w * W
            di = [pltpu.async_copy(src_hbm.at[pl.ds(base, W)], sidx, sem_i),
                  pltpu.async_copy(dst_hbm.at[pl.ds(base, W)], didx, sem_i)]
            for d in di:
                d.wait()
            ds = [pltpu.async_copy(
                tabs[t].at[sidx if idx_sel[t] == 0 else didx], rows[t],
                sem_g) for t in range(nt)]
            for d in ds:
                d.wait()
            os_ = [pltpu.async_copy(rows[t], outs[t].at[pl.ds(base, W)], sem_o)
                   for t in range(nt)]
            for d in os_:
                d.wait()

    return k(*tables, eidx[0], eidx[1])


def _sc_segsum2(sb, sig, dst, N, W=200):
    E, D = sb.shape
    info = plsc.get_sparse_core_info()
    NS = info.num_subcores
    PW = E // NS
    nwin = PW // W
    assert PW % W == 0 and W % 8 == 0
    zeros = jnp.zeros((N, D), F32)
    mesh = plsc.VectorSubcoreMesh(core_axis_name="c", subcore_axis_name="s")

    @functools.partial(
        pl.kernel,
        out_type=jax.ShapeDtypeStruct((2, N, D), F32),
        mesh=mesh,
        scratch_types=[
            pltpu.VMEM((W,), jnp.int32),
            pltpu.VMEM((W, D), F32),
            pltpu.VMEM_SHARED((N, D), F32),
            pltpu.SemaphoreType.DMA,
            pltpu.SemaphoreType.DMA,
        ],
    )
    def k(sb_hbm, sig_hbm, dst_hbm, z_hbm, out_hbm, idxb, upd, acc,
          sem_i, sem_u):
        c = lax.axis_index("c")
        s = lax.axis_index("s")

        @pl.when(s == 0)
        def _():
            pltpu.sync_copy(z_hbm, acc)

        plsc.subcore_barrier()

        @pl.loop(0, nwin)
        def _(w):
            base = s * PW + w * W
            di = pltpu.async_copy(dst_hbm.at[pl.ds(base, W)], idxb, sem_i)

            @pl.when(c == 0)
            def _():
                pltpu.async_copy(sb_hbm.at[pl.ds(base, W)], upd, sem_u).wait()

            @pl.when(c == 1)
            def _():
                pltpu.async_copy(sig_hbm.at[pl.ds(base, W)], upd, sem_u).wait()

            di.wait()
            pltpu.sync_copy(upd, acc.at[idxb], add=True)

        plsc.subcore_barrier()

        @pl.when(s == 0)
        def _():
            pltpu.sync_copy(acc, out_hbm.at[c])

    return k(sb, sig, dst, zeros)


def _pack8(D, *rows):
    a = jnp.stack(rows)
    return jnp.concatenate([a, jnp.zeros((8 - a.shape[0], D), F32)], axis=0)


def kernel(h, e, params, edge_index):
    N, D = h.shape
    src = edge_index[0]
    dst = edge_index[1]
    p1, p2 = params
    eb = 4000

    w1 = jnp.concatenate([p1['A_w'], p1['B_w'], p1['D_w'], p1['E_w']], axis=1)
    b1 = jnp.concatenate([p1['A_b'], p1['B_b'], p1['D_b'], p1['E_b']])
    node1 = _node_mm(h, w1, _pack8(4 * D, b1))
    ah = node1[:, :D]
    bh = node1[:, D:2 * D]
    dh = node1[:, 2 * D:3 * D]
    eh = node1[:, 3 * D:]

    ga1, gb1, bsrc = _sc_gather((dh, eh, bh), (0, 1, 0), edge_index)

    t1, sig, sb, st1 = _edge1(e, ga1, gb1, bsrc, p1['C_w'],
                              _pack8(D, p1['C_b']), eb)

    accs = _sc_segsum2(sb, sig, dst, N)
    num, den = accs[0], accs[1]

    w2 = jnp.concatenate([p2['D_w'], p2['E_w']], axis=1)
    b2 = jnp.concatenate([p2['D_b'], p2['E_b']])
    node2 = _node_fin(h, ah, num, den, _pack8(D, p1['bn_h_g'], p1['bn_h_b']),
                      w2, _pack8(2 * D, b2))
    dh2 = node2[:, :D]
    eh2 = node2[:, D:]

    ga2, gb2 = _sc_gather((dh2, eh2), (0, 1), edge_index)

    pp2 = _pack8(D, p1['bn_e_g'], p1['bn_e_b'], p2['C_b'])
    e1 = _e1pass(e, t1, st1, pp2, eb)
    t2, st2 = _t2pass(e1, ga2, gb2, pp2, p2['C_w'], eb)

    e2 = _edge3(e1, t2, st2, _pack8(D, p2['bn_e_g'], p2['bn_e_b']), eb)

    return (h, e2)


# speedup vs baseline: 1.2468x; 1.0334x over previous
import functools

import jax
import jax.numpy as jnp
from jax import lax
from jax.experimental import pallas as pl
from jax.experimental.pallas import tpu as pltpu
from jax.experimental.pallas import tpu_sc as plsc

F32 = jnp.float32


def _node_mm_body(x_ref, w_ref, b_ref, o_ref):
    o_ref[...] = (jnp.dot(x_ref[...], w_ref[...], preferred_element_type=F32)
                  + b_ref[0:1, :])


def _node_mm(x, w, b8):
    N, _ = x.shape
    K = w.shape[1]
    return pl.pallas_call(
        _node_mm_body,
        out_shape=jax.ShapeDtypeStruct((N, K), F32),
    )(x, w, b8)


def _bfdot(x, w):
    return jnp.dot(x.astype(jnp.bfloat16), w.astype(jnp.bfloat16),
                   preferred_element_type=F32)


def _edge1_body(e_ref, ga_ref, gb_ref, bs_ref, w_ref, p_ref,
                t_ref, sig_ref, sb_ref, st_ref):
    t = (_bfdot(e_ref[...], w_ref[...])
         + p_ref[0:1, :] + ga_ref[...] + gb_ref[...])
    t_ref[...] = t.astype(t_ref.dtype)
    sig = 1.0 / (1.0 + jnp.exp(-t))
    sig_ref[...] = sig
    sb_ref[...] = sig * bs_ref[...]

    @pl.when(pl.program_id(0) == 0)
    def _():
        st_ref[...] = jnp.zeros_like(st_ref)

    st_ref[0:1, :] += jnp.sum(t, axis=0, keepdims=True)
    st_ref[1:2, :] += jnp.sum(t * t, axis=0, keepdims=True)


def _edge1(e0, ga, gb, bsrc, w, p8, eb):
    E, D = e0.shape
    blk = pl.BlockSpec((eb, D), lambda i: (i, 0))
    full = pl.BlockSpec((8, D), lambda i: (0, 0))
    wspec = pl.BlockSpec((D, D), lambda i: (0, 0))
    return pl.pallas_call(
        _edge1_body,
        grid=(E // eb,),
        in_specs=[blk, blk, blk, blk, wspec, full],
        out_specs=[blk, blk, blk, full],
        out_shape=[jax.ShapeDtypeStruct((E, D), jnp.bfloat16),
                   jax.ShapeDtypeStruct((E, D), F32),
                   jax.ShapeDtypeStruct((E, D), F32),
                   jax.ShapeDtypeStruct((8, D), F32)],
    )(e0, ga, gb, bsrc, w, p8)


def _node_fin_body(h0_ref, ah_ref, num_ref, den_ref, pg_ref, w_ref, b_ref, o_ref):
    hpre = ah_ref[...] + num_ref[...] / (den_ref[...] + 1e-6)
    mu = jnp.mean(hpre, axis=0, keepdims=True)
    d = hpre - mu
    var = jnp.mean(d * d, axis=0, keepdims=True)
    hn = pg_ref[0:1, :] * d / jnp.sqrt(var + 1e-5) + pg_ref[1:2, :]
    h1 = h0_ref[...] + jnp.maximum(hn, 0.0)
    o_ref[...] = (jnp.dot(h1, w_ref[...], preferred_element_type=F32)
                  + b_ref[0:1, :])


def _node_fin(h0, ah, num, den, pg8, w2, b2):
    N, _ = h0.shape
    K = w2.shape[1]
    return pl.pallas_call(
        _node_fin_body,
        out_shape=jax.ShapeDtypeStruct((N, K), F32),
    )(h0, ah, num, den, pg8, w2, b2)


def _e1pass_body(inv_e, e0_ref, t1_ref, st1_ref, pp_ref, e1_ref):
    mu = st1_ref[0:1, :] * inv_e
    var = st1_ref[1:2, :] * inv_e - mu * mu
    en = (pp_ref[0:1, :] * (t1_ref[...].astype(F32) - mu) / jnp.sqrt(var + 1e-5)
          + pp_ref[1:2, :])
    e1_ref[...] = (e0_ref[...] + jnp.maximum(en, 0.0)).astype(e1_ref.dtype)


def _e1pass(e0, t1, st1, pp8, eb):
    E, D = e0.shape
    blk = pl.BlockSpec((eb, D), lambda i: (i, 0))
    full = pl.BlockSpec((8, D), lambda i: (0, 0))
    return pl.pallas_call(
        functools.partial(_e1pass_body, 1.0 / E),
        grid=(E // eb,),
        in_specs=[blk, blk, full, full],
        out_specs=blk,
        out_shape=jax.ShapeDtypeStruct((E, D), jnp.bfloat16),
    )(e0, t1, st1, pp8)


def _t2pass_body(e1_ref, ga_ref, gb_ref, pp_ref, w_ref, t2_ref, st2_ref):
    t2 = (_bfdot(e1_ref[...], w_ref[...])
          + pp_ref[2:3, :] + ga_ref[...] + gb_ref[...])
    t2_ref[...] = t2.astype(t2_ref.dtype)

    @pl.when(pl.program_id(0) == 0)
    def _():
        st2_ref[...] = jnp.zeros_like(st2_ref)

    st2_ref[0:1, :] += jnp.sum(t2, axis=0, keepdims=True)
    st2_ref[1:2, :] += jnp.sum(t2 * t2, axis=0, keepdims=True)


def _t2pass(e1, ga, gb, pp8, w, eb):
    E, D = e1.shape
    blk = pl.BlockSpec((eb, D), lambda i: (i, 0))
    full = pl.BlockSpec((8, D), lambda i: (0, 0))
    wspec = pl.BlockSpec((D, D), lambda i: (0, 0))
    return pl.pallas_call(
        _t2pass_body,
        grid=(E // eb,),
        in_specs=[blk, blk, blk, full, wspec],
        out_specs=[blk, full],
        out_shape=[jax.ShapeDtypeStruct((E, D), jnp.bfloat16),
                   jax.ShapeDtypeStruct((8, D), F32)],
    )(e1, ga, gb, pp8, w)


def _edge3_body(inv_e, e1_ref, t2_ref, st2_ref, pp_ref, o_ref):
    mu = st2_ref[0:1, :] * inv_e
    var = st2_ref[1:2, :] * inv_e - mu * mu
    en = (pp_ref[0:1, :] * (t2_ref[...].astype(F32) - mu) / jnp.sqrt(var + 1e-5)
          + pp_ref[1:2, :])
    o_ref[...] = e1_ref[...] + jnp.maximum(en, 0.0)


def _edge3(e1, t2, st2, pp8, eb):
    E, D = e1.shape
    blk = pl.BlockSpec((eb, D), lambda i: (i, 0))
    full = pl.BlockSpec((8, D), lambda i: (0, 0))
    return pl.pallas_call(
        functools.partial(_edge3_body, 1.0 / E),
        grid=(E // eb,),
        in_specs=[blk, blk, full, full],
        out_specs=blk,
        out_shape=jax.ShapeDtypeStruct((E, D), F32),
    )(e1, t2, st2, pp8)


def _sc_gather(tables, idx_sel, eidx, W=None):
    E = eidx[0].shape[0]
    N, D = tables[0].shape
    dt = tables[0].dtype
    nt = len(tables)
    if W is None:
        W = 400 if nt <= 2 else 200
    info = plsc.get_sparse_core_info()
    NC, NS = info.num_cores, info.num_subcores
    NW = NC * NS
    PW = E // NW
    nwin = PW // W
    assert PW % W == 0 and W % 8 == 0
    mesh = plsc.VectorSubcoreMesh(core_axis_name="c", subcore_axis_name="s")

    @functools.partial(
        pl.kernel,
        out_type=[jax.ShapeDtypeStruct((E, D), dt) for _ in range(nt)],
        mesh=mesh,
        scratch_types=(
            [pltpu.VMEM((W,), jnp.int32), pltpu.VMEM((W,), jnp.int32)]
            + [pltpu.VMEM((W, D), dt) for _ in range(nt)]
            + [pltpu.SemaphoreType.DMA, pltpu.SemaphoreType.DMA,
               pltpu.SemaphoreType.DMA]
        ),
    )
    def k(*refs):
        tabs = refs[:nt]
        src_hbm, dst_hbm = refs[nt], refs[nt + 1]
        outs = refs[nt + 2:2 * nt + 2]
        sidx, didx = refs[2 * nt + 2], refs[2 * nt + 3]
        rows = refs[2 * nt + 4:3 * nt + 4]
        sem_i, sem_g, sem_o = refs[3 * nt + 4:3 * nt + 7]
        wid = lax.axis_index("s") * NC + lax.axis_index("c")

        @pl.loop(0, nwin)
        def _(w):
            base = wid * PW + w * W
            di = [pltpu.async_copy(src_hbm.at[pl.ds(base, W)], sidx, sem_i),
                  pltpu.async_copy(dst_hbm.at[pl.ds(base, W)], didx, sem_i)]
            for d in di:
                d.wait()
            ds = [pltpu.async_copy(
                tabs[t].at[sidx if idx_sel[t] == 0 else didx], rows[t],
                sem_g) for t in range(nt)]
            for d in ds:
                d.wait()
            os_ = [pltpu.async_copy(rows[t], outs[t].at[pl.ds(base, W)], sem_o)
                   for t in range(nt)]
            for d in os_:
                d.wait()

    return k(*tables, eidx[0], eidx[1])


def _sc_segsum2(sb, sig, dst, N, W=200):
    E, D = sb.shape
    info = plsc.get_sparse_core_info()
    NS = info.num_subcores
    PW = E // NS
    nwin = PW // W
    assert PW % W == 0 and W % 8 == 0
    zeros = jnp.zeros((N, D), F32)
    mesh = plsc.VectorSubcoreMesh(core_axis_name="c", subcore_axis_name="s")

    @functools.partial(
        pl.kernel,
        out_type=jax.ShapeDtypeStruct((2, N, D), F32),
        mesh=mesh,
        scratch_types=[
            pltpu.VMEM((W,), jnp.int32),
            pltpu.VMEM((W, D), F32),
            pltpu.VMEM_SHARED((N, D), F32),
            pltpu.SemaphoreType.DMA,
            pltpu.SemaphoreType.DMA,
        ],
    )
    def k(sb_hbm, sig_hbm, dst_hbm, z_hbm, out_hbm, idxb, upd, acc,
          sem_i, sem_u):
        c = lax.axis_index("c")
        s = lax.axis_index("s")

        @pl.when(s == 0)
        def _():
            pltpu.sync_copy(z_hbm, acc)

        plsc.subcore_barrier()

        @pl.loop(0, nwin)
        def _(w):
            base = s * PW + w * W
            di = pltpu.async_copy(dst_hbm.at[pl.ds(base, W)], idxb, sem_i)

            @pl.when(c == 0)
            def _():
                pltpu.async_copy(sb_hbm.at[pl.ds(base, W)], upd, sem_u).wait()

            @pl.when(c == 1)
            def _():
                pltpu.async_copy(sig_hbm.at[pl.ds(base, W)], upd, sem_u).wait()

            di.wait()
            pltpu.sync_copy(upd, acc.at[idxb], add=True)

        plsc.subcore_barrier()

        @pl.when(s == 0)
        def _():
            pltpu.sync_copy(acc, out_hbm.at[c])

    return k(sb, sig, dst, zeros)


def _pack8(D, *rows):
    a = jnp.stack(rows)
    return jnp.concatenate([a, jnp.zeros((8 - a.shape[0], D), F32)], axis=0)


def kernel(h, e, params, edge_index):
    N, D = h.shape
    src = edge_index[0]
    dst = edge_index[1]
    p1, p2 = params
    eb = 4000

    w1 = jnp.concatenate([p1['A_w'], p1['B_w'], p1['D_w'], p1['E_w']], axis=1)
    b1 = jnp.concatenate([p1['A_b'], p1['B_b'], p1['D_b'], p1['E_b']])
    node1 = _node_mm(h, w1, _pack8(4 * D, b1))
    ah = node1[:, :D]
    bh = node1[:, D:2 * D]
    dh = node1[:, 2 * D:3 * D]
    eh = node1[:, 3 * D:]

    ga1, gb1, bsrc = _sc_gather((dh, eh, bh), (0, 1, 0), edge_index)

    t1, sig, sb, st1 = _edge1(e, ga1, gb1, bsrc, p1['C_w'],
                              _pack8(D, p1['C_b']), eb)

    accs = _sc_segsum2(sb, sig, dst, N)
    num, den = accs[0], accs[1]

    w2 = jnp.concatenate([p2['D_w'], p2['E_w']], axis=1)
    b2 = jnp.concatenate([p2['D_b'], p2['E_b']])
    node2 = _node_fin(h, ah, num, den, _pack8(D, p1['bn_h_g'], p1['bn_h_b']),
                      w2, _pack8(2 * D, b2))
    dh2 = node2[:, :D]
    eh2 = node2[:, D:]

    ga2, gb2 = _sc_gather((dh2, eh2), (0, 1), edge_index)

    pp2 = _pack8(D, p1['bn_e_g'], p1['bn_e_b'], p2['C_b'])
    e1 = _e1pass(e, t1, st1, pp2, eb)
    t2, st2 = _t2pass(e1, ga2, gb2, pp2, p2['C_w'], eb)

    e2 = _edge3(e1, t2, st2, _pack8(D, p2['bn_e_g'], p2['bn_e_b']), eb)

    return (h, e2)


# speedup vs baseline: 1.2607x; 1.0111x over previous
import functools

import jax
import jax.numpy as jnp
from jax import lax
from jax.experimental import pallas as pl
from jax.experimental.pallas import tpu as pltpu
from jax.experimental.pallas import tpu_sc as plsc

F32 = jnp.float32


def _node_mm_body(x_ref, w_ref, b_ref, o_ref):
    o_ref[...] = (jnp.dot(x_ref[...], w_ref[...], preferred_element_type=F32)
                  + b_ref[0:1, :])


def _node_mm(x, w, b8):
    N, _ = x.shape
    K = w.shape[1]
    return pl.pallas_call(
        _node_mm_body,
        out_shape=jax.ShapeDtypeStruct((N, K), F32),
    )(x, w, b8)


def _bfdot(x, w):
    return jnp.dot(x.astype(jnp.bfloat16), w.astype(jnp.bfloat16),
                   preferred_element_type=F32)


def _edge1_body(e_ref, ga_ref, gb_ref, bs_ref, w_ref, p_ref,
                t_ref, sig_ref, sb_ref, st_ref):
    t = (_bfdot(e_ref[...], w_ref[...])
         + p_ref[0:1, :] + ga_ref[...] + gb_ref[...])
    t_ref[...] = t.astype(t_ref.dtype)
    sig = 1.0 / (1.0 + jnp.exp(-t))
    sig_ref[...] = sig
    sb_ref[...] = sig * bs_ref[...]

    @pl.when(pl.program_id(0) == 0)
    def _():
        st_ref[...] = jnp.zeros_like(st_ref)

    st_ref[0:1, :] += jnp.sum(t, axis=0, keepdims=True)
    st_ref[1:2, :] += jnp.sum(t * t, axis=0, keepdims=True)


def _edge1_acc_body(e_ref, ga_ref, gb_ref, bs_ref, w_ref, p_ref,
                    t_in, sg_in, sb_in, st_in, t_ref, sig_ref, sb_ref, st_ref):
    t = (_bfdot(e_ref[...], w_ref[...])
         + p_ref[0:1, :] + ga_ref[...] + gb_ref[...])
    t_ref[...] = t.astype(t_ref.dtype)
    sig = 1.0 / (1.0 + jnp.exp(-t))
    sig_ref[...] = sig
    sb_ref[...] = sig * bs_ref[...]

    @pl.when(pl.program_id(0) == 0)
    def _():
        st_ref[...] = st_in[...]

    st_ref[0:1, :] += jnp.sum(t, axis=0, keepdims=True)
    st_ref[1:2, :] += jnp.sum(t * t, axis=0, keepdims=True)


def _edge1(e0, ga, gb, bsrc, w, p8, eb, half, prev=None):
    E, D = e0.shape
    EH = E // 2
    nb = EH // eb
    off = half * nb
    blk_full = pl.BlockSpec((eb, D), lambda i: (i + off, 0))
    blk_half = pl.BlockSpec((eb, D), lambda i: (i, 0))
    full = pl.BlockSpec((8, D), lambda i: (0, 0))
    wspec = pl.BlockSpec((D, D), lambda i: (0, 0))
    out_specs = [blk_full, blk_full, blk_full, full]
    out_shape = [jax.ShapeDtypeStruct((E, D), jnp.bfloat16),
                 jax.ShapeDtypeStruct((E, D), F32),
                 jax.ShapeDtypeStruct((E, D), F32),
                 jax.ShapeDtypeStruct((8, D), F32)]
    if half == 0:
        return pl.pallas_call(
            _edge1_body,
            grid=(nb,),
            in_specs=[blk_full, blk_half, blk_half, blk_half, wspec, full],
            out_specs=out_specs,
            out_shape=out_shape,
        )(e0, ga, gb, bsrc, w, p8)
    return pl.pallas_call(
        _edge1_acc_body,
        grid=(nb,),
        in_specs=[blk_full, blk_half, blk_half, blk_half, wspec, full,
                  full, full, full, full],
        out_specs=out_specs,
        out_shape=out_shape,
        input_output_aliases={6: 0, 7: 1, 8: 2, 9: 3},
    )(e0, ga, gb, bsrc, w, p8, *prev)


def _node_fin_body(h0_ref, ah_ref, num_ref, den_ref, pg_ref, w_ref, b_ref, o_ref):
    hpre = ah_ref[...] + num_ref[...] / (den_ref[...] + 1e-6)
    mu = jnp.mean(hpre, axis=0, keepdims=True)
    d = hpre - mu
    var = jnp.mean(d * d, axis=0, keepdims=True)
    hn = pg_ref[0:1, :] * d / jnp.sqrt(var + 1e-5) + pg_ref[1:2, :]
    h1 = h0_ref[...] + jnp.maximum(hn, 0.0)
    o_ref[...] = (jnp.dot(h1, w_ref[...], preferred_element_type=F32)
                  + b_ref[0:1, :])


def _node_fin(h0, ah, num, den, pg8, w2, b2):
    N, _ = h0.shape
    K = w2.shape[1]
    return pl.pallas_call(
        _node_fin_body,
        out_shape=jax.ShapeDtypeStruct((N, K), F32),
    )(h0, ah, num, den, pg8, w2, b2)


def _e1pass_body(inv_e, e0_ref, t1_ref, st1_ref, pp_ref, e1_ref):
    mu = st1_ref[0:1, :] * inv_e
    var = st1_ref[1:2, :] * inv_e - mu * mu
    en = (pp_ref[0:1, :] * (t1_ref[...].astype(F32) - mu) / jnp.sqrt(var + 1e-5)
          + pp_ref[1:2, :])
    e1_ref[...] = (e0_ref[...] + jnp.maximum(en, 0.0)).astype(e1_ref.dtype)


def _e1pass(e0, t1, st1, pp8, eb):
    E, D = e0.shape
    blk = pl.BlockSpec((eb, D), lambda i: (i, 0))
    full = pl.BlockSpec((8, D), lambda i: (0, 0))
    return pl.pallas_call(
        functools.partial(_e1pass_body, 1.0 / E),
        grid=(E // eb,),
        in_specs=[blk, blk, full, full],
        out_specs=blk,
        out_shape=jax.ShapeDtypeStruct((E, D), jnp.bfloat16),
    )(e0, t1, st1, pp8)


def _t2pass_body(e1_ref, ga_ref, gb_ref, pp_ref, w_ref, t2_ref, st2_ref):
    t2 = (_bfdot(e1_ref[...], w_ref[...])
          + pp_ref[2:3, :] + ga_ref[...] + gb_ref[...])
    t2_ref[...] = t2.astype(t2_ref.dtype)

    @pl.when(pl.program_id(0) == 0)
    def _():
        st2_ref[...] = jnp.zeros_like(st2_ref)

    st2_ref[0:1, :] += jnp.sum(t2, axis=0, keepdims=True)
    st2_ref[1:2, :] += jnp.sum(t2 * t2, axis=0, keepdims=True)


def _t2pass(e1, ga, gb, pp8, w, eb):
    E, D = e1.shape
    blk = pl.BlockSpec((eb, D), lambda i: (i, 0))
    full = pl.BlockSpec((8, D), lambda i: (0, 0))
    wspec = pl.BlockSpec((D, D), lambda i: (0, 0))
    return pl.pallas_call(
        _t2pass_body,
        grid=(E // eb,),
        in_specs=[blk, blk, blk, full, wspec],
        out_specs=[blk, full],
        out_shape=[jax.ShapeDtypeStruct((E, D), jnp.bfloat16),
                   jax.ShapeDtypeStruct((8, D), F32)],
    )(e1, ga, gb, pp8, w)


def _edge3_body(inv_e, e1_ref, t2_ref, st2_ref, pp_ref, o_ref):
    mu = st2_ref[0:1, :] * inv_e
    var = st2_ref[1:2, :] * inv_e - mu * mu
    en = (pp_ref[0:1, :] * (t2_ref[...].astype(F32) - mu) / jnp.sqrt(var + 1e-5)
          + pp_ref[1:2, :])
    o_ref[...] = e1_ref[...] + jnp.maximum(en, 0.0)


def _edge3(e1, t2, st2, pp8, eb):
    E, D = e1.shape
    blk = pl.BlockSpec((eb, D), lambda i: (i, 0))
    full = pl.BlockSpec((8, D), lambda i: (0, 0))
    return pl.pallas_call(
        functools.partial(_edge3_body, 1.0 / E),
        grid=(E // eb,),
        in_specs=[blk, blk, full, full],
        out_specs=blk,
        out_shape=jax.ShapeDtypeStruct((E, D), F32),
    )(e1, t2, st2, pp8)


def _sc_gather(tables, idx_sel, eidx, lo=0, cnt=None, W=None):
    E = eidx[0].shape[0]
    if cnt is None:
        cnt = E
    N, D = tables[0].shape
    dt = tables[0].dtype
    nt = len(tables)
    info = plsc.get_sparse_core_info()
    NC, NS = info.num_cores, info.num_subcores
    NW = NC * NS
    PW = cnt // NW
    if W is None:
        W = 400 if (nt <= 2 and PW % 400 == 0) else 200
    nwin = PW // W
    assert PW % W == 0 and W % 8 == 0 and lo % 8 == 0
    mesh = plsc.VectorSubcoreMesh(core_axis_name="c", subcore_axis_name="s")

    @functools.partial(
        pl.kernel,
        out_type=[jax.ShapeDtypeStruct((cnt, D), dt) for _ in range(nt)],
        mesh=mesh,
        scratch_types=(
            [pltpu.VMEM((W,), jnp.int32), pltpu.VMEM((W,), jnp.int32)]
            + [pltpu.VMEM((W, D), dt) for _ in range(nt)]
            + [pltpu.SemaphoreType.DMA, pltpu.SemaphoreType.DMA,
               pltpu.SemaphoreType.DMA]
        ),
    )
    def k(*refs):
        tabs = refs[:nt]
        src_hbm, dst_hbm = refs[nt], refs[nt + 1]
        outs = refs[nt + 2:2 * nt + 2]
        sidx, didx = refs[2 * nt + 2], refs[2 * nt + 3]
        rows = refs[2 * nt + 4:3 * nt + 4]
        sem_i, sem_g, sem_o = refs[3 * nt + 4:3 * nt + 7]
        wid = lax.axis_index("s") * NC + lax.axis_index("c")

        @pl.loop(0, nwin)
        def _(w):
            base = wid * PW + w * W
            di = [pltpu.async_copy(src_hbm.at[pl.ds(lo + base, W)], sidx,
                                   sem_i),
                  pltpu.async_copy(dst_hbm.at[pl.ds(lo + base, W)], didx,
                                   sem_i)]
            for d in di:
                d.wait()
            ds = [pltpu.async_copy(
                tabs[t].at[sidx if idx_sel[t] == 0 else didx], rows[t],
                sem_g) for t in range(nt)]
            for d in ds:
                d.wait()
            os_ = [pltpu.async_copy(rows[t], outs[t].at[pl.ds(base, W)], sem_o)
                   for t in range(nt)]
            for d in os_:
                d.wait()

    return k(*tables, eidx[0], eidx[1])


def _sc_segsum2(sb, sig, dst, N, W=200):
    E, D = sb.shape
    info = plsc.get_sparse_core_info()
    NS = info.num_subcores
    PW = E // NS
    nwin = PW // W
    assert PW % W == 0 and W % 8 == 0
    zeros = jnp.zeros((N, D), F32)
    mesh = plsc.VectorSubcoreMesh(core_axis_name="c", subcore_axis_name="s")

    @functools.partial(
        pl.kernel,
        out_type=jax.ShapeDtypeStruct((2, N, D), F32),
        mesh=mesh,
        scratch_types=[
            pltpu.VMEM((W,), jnp.int32),
            pltpu.VMEM((W, D), F32),
            pltpu.VMEM_SHARED((N, D), F32),
            pltpu.SemaphoreType.DMA,
            pltpu.SemaphoreType.DMA,
        ],
    )
    def k(sb_hbm, sig_hbm, dst_hbm, z_hbm, out_hbm, idxb, upd, acc,
          sem_i, sem_u):
        c = lax.axis_index("c")
        s = lax.axis_index("s")

        @pl.when(s == 0)
        def _():
            pltpu.sync_copy(z_hbm, acc)

        plsc.subcore_barrier()

        @pl.loop(0, nwin)
        def _(w):
            base = s * PW + w * W
            di = pltpu.async_copy(dst_hbm.at[pl.ds(base, W)], idxb, sem_i)

            @pl.when(c == 0)
            def _():
                pltpu.async_copy(sb_hbm.at[pl.ds(base, W)], upd, sem_u).wait()

            @pl.when(c == 1)
            def _():
                pltpu.async_copy(sig_hbm.at[pl.ds(base, W)], upd, sem_u).wait()

            di.wait()
            pltpu.sync_copy(upd, acc.at[idxb], add=True)

        plsc.subcore_barrier()

        @pl.when(s == 0)
        def _():
            pltpu.sync_copy(acc, out_hbm.at[c])

    return k(sb, sig, dst, zeros)


def _pack8(D, *rows):
    a = jnp.stack(rows)
    return jnp.concatenate([a, jnp.zeros((8 - a.shape[0], D), F32)], axis=0)


def kernel(h, e, params, edge_index):
    N, D = h.shape
    src = edge_index[0]
    dst = edge_index[1]
    p1, p2 = params
    eb = 4000

    w1 = jnp.concatenate([p1['A_w'], p1['B_w'], p1['D_w'], p1['E_w']], axis=1)
    b1 = jnp.concatenate([p1['A_b'], p1['B_b'], p1['D_b'], p1['E_b']])
    node1 = _node_mm(h, w1, _pack8(4 * D, b1))
    ah = node1[:, :D]
    bh = node1[:, D:2 * D]
    dh = node1[:, 2 * D:3 * D]
    eh = node1[:, 3 * D:]

    E = e.shape[0]
    ga1a, gb1a, bsa = _sc_gather((dh, eh, bh), (0, 1, 0), edge_index,
                                 0, E // 2)
    ga1b, gb1b, bsb = _sc_gather((dh, eh, bh), (0, 1, 0), edge_index,
                                 E // 2, E // 2)

    p8c1 = _pack8(D, p1['C_b'])
    prev = _edge1(e, ga1a, gb1a, bsa, p1['C_w'], p8c1, eb, 0)
    t1, sig, sb, st1 = _edge1(e, ga1b, gb1b, bsb, p1['C_w'], p8c1, eb, 1,
                              prev)

    accs = _sc_segsum2(sb, sig, dst, N)
    num, den = accs[0], accs[1]

    w2 = jnp.concatenate([p2['D_w'], p2['E_w']], axis=1)
    b2 = jnp.concatenate([p2['D_b'], p2['E_b']])
    node2 = _node_fin(h, ah, num, den, _pack8(D, p1['bn_h_g'], p1['bn_h_b']),
                      w2, _pack8(2 * D, b2))
    dh2 = node2[:, :D]
    eh2 = node2[:, D:]

    ga2, gb2 = _sc_gather((dh2, eh2), (0, 1), edge_index)

    pp2 = _pack8(D, p1['bn_e_g'], p1['bn_e_b'], p2['C_b'])
    e1 = _e1pass(e, t1, st1, pp2, eb)
    t2, st2 = _t2pass(e1, ga2, gb2, pp2, p2['C_w'], eb)

    e2 = _edge3(e1, t2, st2, _pack8(D, p2['bn_e_g'], p2['bn_e_b']), eb)

    return (h, e2)


# speedup vs baseline: 1.2838x; 1.0183x over previous
import functools

import jax
import jax.numpy as jnp
from jax import lax
from jax.experimental import pallas as pl
from jax.experimental.pallas import tpu as pltpu
from jax.experimental.pallas import tpu_sc as plsc

F32 = jnp.float32


def _node_mm_body(x_ref, w_ref, b_ref, o_ref):
    o_ref[...] = (jnp.dot(x_ref[...], w_ref[...], preferred_element_type=F32)
                  + b_ref[0:1, :])


def _node_mm(x, w, b8):
    N, _ = x.shape
    K = w.shape[1]
    return pl.pallas_call(
        _node_mm_body,
        out_shape=jax.ShapeDtypeStruct((N, K), F32),
    )(x, w, b8)


def _bfdot(x, w):
    return jnp.dot(x.astype(jnp.bfloat16), w.astype(jnp.bfloat16),
                   preferred_element_type=F32)


def _edge1_body(e_ref, ga_ref, gb_ref, bs_ref, w_ref, p_ref,
                t_ref, sig_ref, sb_ref, st_ref):
    t = (_bfdot(e_ref[...], w_ref[...])
         + p_ref[0:1, :] + ga_ref[...] + gb_ref[...])
    t_ref[...] = t.astype(t_ref.dtype)
    sig = 1.0 / (1.0 + jnp.exp(-t))
    sig_ref[...] = sig
    sb_ref[...] = sig * bs_ref[...]

    @pl.when(pl.program_id(0) == 0)
    def _():
        st_ref[...] = jnp.zeros_like(st_ref)

    st_ref[0:1, :] += jnp.sum(t, axis=0, keepdims=True)
    st_ref[1:2, :] += jnp.sum(t * t, axis=0, keepdims=True)


def _edge1_acc_body(e_ref, ga_ref, gb_ref, bs_ref, w_ref, p_ref,
                    t_in, sg_in, sb_in, st_in, t_ref, sig_ref, sb_ref, st_ref):
    t = (_bfdot(e_ref[...], w_ref[...])
         + p_ref[0:1, :] + ga_ref[...] + gb_ref[...])
    t_ref[...] = t.astype(t_ref.dtype)
    sig = 1.0 / (1.0 + jnp.exp(-t))
    sig_ref[...] = sig
    sb_ref[...] = sig * bs_ref[...]

    @pl.when(pl.program_id(0) == 0)
    def _():
        st_ref[...] = st_in[...]

    st_ref[0:1, :] += jnp.sum(t, axis=0, keepdims=True)
    st_ref[1:2, :] += jnp.sum(t * t, axis=0, keepdims=True)


def _edge1(e0, ga, gb, bsrc, w, p8, eb, half, prev=None):
    E, D = e0.shape
    EH = E // 2
    nb = EH // eb
    off = half * nb
    blk_full = pl.BlockSpec((eb, D), lambda i: (i + off, 0))
    blk_half = pl.BlockSpec((eb, D), lambda i: (i, 0))
    full = pl.BlockSpec((8, D), lambda i: (0, 0))
    wspec = pl.BlockSpec((D, D), lambda i: (0, 0))
    out_specs = [blk_full, blk_full, blk_full, full]
    out_shape = [jax.ShapeDtypeStruct((E, D), jnp.bfloat16),
                 jax.ShapeDtypeStruct((E, D), F32),
                 jax.ShapeDtypeStruct((E, D), F32),
                 jax.ShapeDtypeStruct((8, D), F32)]
    if half == 0:
        return pl.pallas_call(
            _edge1_body,
            grid=(nb,),
            in_specs=[blk_full, blk_half, blk_half, blk_half, wspec, full],
            out_specs=out_specs,
            out_shape=out_shape,
        )(e0, ga, gb, bsrc, w, p8)
    return pl.pallas_call(
        _edge1_acc_body,
        grid=(nb,),
        in_specs=[blk_full, blk_half, blk_half, blk_half, wspec, full,
                  full, full, full, full],
        out_specs=out_specs,
        out_shape=out_shape,
        input_output_aliases={6: 0, 7: 1, 8: 2, 9: 3},
    )(e0, ga, gb, bsrc, w, p8, *prev)


def _node_fin_body(h0_ref, ah_ref, num_ref, den_ref, pg_ref, w_ref, b_ref, o_ref):
    hpre = ah_ref[...] + num_ref[...] / (den_ref[...] + 1e-6)
    mu = jnp.mean(hpre, axis=0, keepdims=True)
    d = hpre - mu
    var = jnp.mean(d * d, axis=0, keepdims=True)
    hn = pg_ref[0:1, :] * d / jnp.sqrt(var + 1e-5) + pg_ref[1:2, :]
    h1 = h0_ref[...] + jnp.maximum(hn, 0.0)
    o_ref[...] = (jnp.dot(h1, w_ref[...], preferred_element_type=F32)
                  + b_ref[0:1, :])


def _node_fin(h0, ah, num, den, pg8, w2, b2):
    N, _ = h0.shape
    K = w2.shape[1]
    return pl.pallas_call(
        _node_fin_body,
        out_shape=jax.ShapeDtypeStruct((N, K), F32),
    )(h0, ah, num, den, pg8, w2, b2)


def _e1pass_body(inv_e, e0_ref, t1_ref, st1_ref, pp_ref, e1_ref):
    mu = st1_ref[0:1, :] * inv_e
    var = st1_ref[1:2, :] * inv_e - mu * mu
    en = (pp_ref[0:1, :] * (t1_ref[...].astype(F32) - mu) / jnp.sqrt(var + 1e-5)
          + pp_ref[1:2, :])
    e1_ref[...] = (e0_ref[...] + jnp.maximum(en, 0.0)).astype(e1_ref.dtype)


def _e1pass(e0, t1, st1, pp8, eb):
    E, D = e0.shape
    blk = pl.BlockSpec((eb, D), lambda i: (i, 0))
    full = pl.BlockSpec((8, D), lambda i: (0, 0))
    return pl.pallas_call(
        functools.partial(_e1pass_body, 1.0 / E),
        grid=(E // eb,),
        in_specs=[blk, blk, full, full],
        out_specs=blk,
        out_shape=jax.ShapeDtypeStruct((E, D), jnp.bfloat16),
    )(e0, t1, st1, pp8)


def _t2pass_body(e1_ref, ga_ref, gb_ref, pp_ref, w_ref, t2_ref, st2_ref):
    t2 = (_bfdot(e1_ref[...], w_ref[...])
          + pp_ref[2:3, :] + ga_ref[...] + gb_ref[...])
    t2_ref[...] = t2.astype(t2_ref.dtype)

    @pl.when(pl.program_id(0) == 0)
    def _():
        st2_ref[...] = jnp.zeros_like(st2_ref)

    st2_ref[0:1, :] += jnp.sum(t2, axis=0, keepdims=True)
    st2_ref[1:2, :] += jnp.sum(t2 * t2, axis=0, keepdims=True)


def _t2pass(e1, ga, gb, pp8, w, eb):
    E, D = e1.shape
    blk = pl.BlockSpec((eb, D), lambda i: (i, 0))
    full = pl.BlockSpec((8, D), lambda i: (0, 0))
    wspec = pl.BlockSpec((D, D), lambda i: (0, 0))
    return pl.pallas_call(
        _t2pass_body,
        grid=(E // eb,),
        in_specs=[blk, blk, blk, full, wspec],
        out_specs=[blk, full],
        out_shape=[jax.ShapeDtypeStruct((E, D), jnp.bfloat16),
                   jax.ShapeDtypeStruct((8, D), F32)],
    )(e1, ga, gb, pp8, w)


def _edge3_body(inv_e, e1_ref, t2_ref, st2_ref, pp_ref, o_ref):
    mu = st2_ref[0:1, :] * inv_e
    var = st2_ref[1:2, :] * inv_e - mu * mu
    en = (pp_ref[0:1, :] * (t2_ref[...].astype(F32) - mu) / jnp.sqrt(var + 1e-5)
          + pp_ref[1:2, :])
    o_ref[...] = e1_ref[...] + jnp.maximum(en, 0.0)


def _edge3(e1, t2, st2, pp8, eb):
    E, D = e1.shape
    blk = pl.BlockSpec((eb, D), lambda i: (i, 0))
    full = pl.BlockSpec((8, D), lambda i: (0, 0))
    return pl.pallas_call(
        functools.partial(_edge3_body, 1.0 / E),
        grid=(E // eb,),
        in_specs=[blk, blk, full, full],
        out_specs=blk,
        out_shape=jax.ShapeDtypeStruct((E, D), F32),
    )(e1, t2, st2, pp8)


def _sc_gather(tables, idx_sel, eidx, lo=0, cnt=None, W=None):
    E = eidx[0].shape[0]
    if cnt is None:
        cnt = E
    N, D = tables[0].shape
    dt = tables[0].dtype
    nt = len(tables)
    info = plsc.get_sparse_core_info()
    NC, NS = info.num_cores, info.num_subcores
    NW = NC * NS
    PW = cnt // NW
    if W is None:
        W = 400 if (nt <= 2 and PW % 400 == 0) else 200
    nwin = PW // W
    assert PW % W == 0 and W % 8 == 0 and lo % 8 == 0
    mesh = plsc.VectorSubcoreMesh(core_axis_name="c", subcore_axis_name="s")

    @functools.partial(
        pl.kernel,
        out_type=[jax.ShapeDtypeStruct((cnt, D), dt) for _ in range(nt)],
        mesh=mesh,
        scratch_types=(
            [pltpu.VMEM((W,), jnp.int32), pltpu.VMEM((W,), jnp.int32)]
            + [pltpu.VMEM((W, D), dt) for _ in range(nt)]
            + [pltpu.SemaphoreType.DMA, pltpu.SemaphoreType.DMA,
               pltpu.SemaphoreType.DMA]
        ),
    )
    def k(*refs):
        tabs = refs[:nt]
        src_hbm, dst_hbm = refs[nt], refs[nt + 1]
        outs = refs[nt + 2:2 * nt + 2]
        sidx, didx = refs[2 * nt + 2], refs[2 * nt + 3]
        rows = refs[2 * nt + 4:3 * nt + 4]
        sem_i, sem_g, sem_o = refs[3 * nt + 4:3 * nt + 7]
        wid = lax.axis_index("s") * NC + lax.axis_index("c")

        @pl.loop(0, nwin)
        def _(w):
            base = wid * PW + w * W
            di = [pltpu.async_copy(src_hbm.at[pl.ds(lo + base, W)], sidx,
                                   sem_i),
                  pltpu.async_copy(dst_hbm.at[pl.ds(lo + base, W)], didx,
                                   sem_i)]
            for d in di:
                d.wait()
            ds = [pltpu.async_copy(
                tabs[t].at[sidx if idx_sel[t] == 0 else didx], rows[t],
                sem_g) for t in range(nt)]
            for d in ds:
                d.wait()
            os_ = [pltpu.async_copy(rows[t], outs[t].at[pl.ds(base, W)], sem_o)
                   for t in range(nt)]
            for d in os_:
                d.wait()

    return k(*tables, eidx[0], eidx[1])


def _sc_segsum2(sb, sig, dst, N, W=200):
    E, D = sb.shape
    info = plsc.get_sparse_core_info()
    NS = info.num_subcores
    PW = E // NS
    nwin = PW // W
    assert PW % W == 0 and W % 8 == 0
    zeros = jnp.zeros((N, D), F32)
    mesh = plsc.VectorSubcoreMesh(core_axis_name="c", subcore_axis_name="s")

    @functools.partial(
        pl.kernel,
        out_type=jax.ShapeDtypeStruct((2, N, D), F32),
        mesh=mesh,
        scratch_types=[
            pltpu.VMEM((W,), jnp.int32),
            pltpu.VMEM((W, D), F32),
            pltpu.VMEM_SHARED((N, D), F32),
            pltpu.SemaphoreType.DMA,
            pltpu.SemaphoreType.DMA,
        ],
    )
    def k(sb_hbm, sig_hbm, dst_hbm, z_hbm, out_hbm, idxb, upd, acc,
          sem_i, sem_u):
        c = lax.axis_index("c")
        s = lax.axis_index("s")

        @pl.when(s == 0)
        def _():
            pltpu.sync_copy(z_hbm, acc)

        plsc.subcore_barrier()

        @pl.loop(0, nwin)
        def _(w):
            base = s * PW + w * W
            di = pltpu.async_copy(dst_hbm.at[pl.ds(base, W)], idxb, sem_i)

            @pl.when(c == 0)
            def _():
                pltpu.async_copy(sb_hbm.at[pl.ds(base, W)], upd, sem_u).wait()

            @pl.when(c == 1)
            def _():
                pltpu.async_copy(sig_hbm.at[pl.ds(base, W)], upd, sem_u).wait()

            di.wait()
            pltpu.sync_copy(upd, acc.at[idxb], add=True)

        plsc.subcore_barrier()

        @pl.when(s == 0)
        def _():
            pltpu.sync_copy(acc, out_hbm.at[c])

    return k(sb, sig, dst, zeros)


def _pack8(D, *rows):
    a = jnp.stack(rows)
    return jnp.concatenate([a, jnp.zeros((8 - a.shape[0], D), F32)], axis=0)


def kernel(h, e, params, edge_index):
    N, D = h.shape
    src = edge_index[0]
    dst = edge_index[1]
    p1, p2 = params
    eb = 8000

    w1 = jnp.concatenate([p1['A_w'], p1['B_w'], p1['D_w'], p1['E_w']], axis=1)
    b1 = jnp.concatenate([p1['A_b'], p1['B_b'], p1['D_b'], p1['E_b']])
    node1 = _node_mm(h, w1, _pack8(4 * D, b1))
    ah = node1[:, :D]
    bh = node1[:, D:2 * D]
    dh = node1[:, 2 * D:3 * D]
    eh = node1[:, 3 * D:]

    E = e.shape[0]
    ga1a, gb1a, bsa = _sc_gather((dh, eh, bh), (0, 1, 0), edge_index,
                                 0, E // 2)
    ga1b, gb1b, bsb = _sc_gather((dh, eh, bh), (0, 1, 0), edge_index,
                                 E // 2, E // 2)

    p8c1 = _pack8(D, p1['C_b'])
    prev = _edge1(e, ga1a, gb1a, bsa, p1['C_w'], p8c1, eb, 0)
    t1, sig, sb, st1 = _edge1(e, ga1b, gb1b, bsb, p1['C_w'], p8c1, eb, 1,
                              prev)

    accs = _sc_segsum2(sb, sig, dst, N)
    num, den = accs[0], accs[1]

    w2 = jnp.concatenate([p2['D_w'], p2['E_w']], axis=1)
    b2 = jnp.concatenate([p2['D_b'], p2['E_b']])
    node2 = _node_fin(h, ah, num, den, _pack8(D, p1['bn_h_g'], p1['bn_h_b']),
                      w2, _pack8(2 * D, b2))
    dh2 = node2[:, :D]
    eh2 = node2[:, D:]

    ga2, gb2 = _sc_gather((dh2, eh2), (0, 1), edge_index)

    pp2 = _pack8(D, p1['bn_e_g'], p1['bn_e_b'], p2['C_b'])
    e1 = _e1pass(e, t1, st1, pp2, eb)
    t2, st2 = _t2pass(e1, ga2, gb2, pp2, p2['C_w'], eb)

    e2 = _edge3(e1, t2, st2, _pack8(D, p2['bn_e_g'], p2['bn_e_b']), eb)

    return (h, e2)
